# baseline (speedup 1.0000x reference)
import functools

import jax
import jax.numpy as jnp
from jax import lax
from jax.experimental import pallas as pl
from jax.experimental.pallas import tpu as pltpu
from jax.experimental.pallas import tpu_sc as plsc

_N = 10000
_E = 320000
_ND = 128
_HD = 256
_ED = 16
_NRBF = 16
_CUTOFF = 10.0
_XP = 16

_S = 2
_ES = _E // _S
_CH = 128
_NCHS = _ES // _CH
_NC = 2
_NS = 16
_NW = _NC * _NS
_NPS = _N // _NS

_BE = 1600
_BN = 1000

_GTRIPS = (_NCHS + _NW - 1) // _NW
_GPAIRS = (_GTRIPS + 1) // 2
_STRIPS = (_NCHS + _NS - 1) // _NS
_SPAIRS = (_STRIPS + 1) // 2


@functools.lru_cache(maxsize=None)
def _sc_mesh():
    return plsc.VectorSubcoreMesh(
        core_axis_name="c", subcore_axis_name="s", num_cores=_NC, num_subcores=_NS
    )


def _silu(v):
    return v * jax.nn.sigmoid(v)


def _mk_gather_pipe(h_hbm, xp_hbm, ii_hbm, jj_hbm,
                    hi_hbm, hj_hbm, xi_hbm, xj_hbm, bufs, wid):
    def valid(t):
        return (t >= 0) & (wid + t * _NW < _NCHS)

    def fire_gather(t, b):
        ii_v, jj_v, bhi, bhj, bxi, bxj, gsem, _ = bufs[b]

        @pl.when(valid(t))
        def _():
            base = (wid + t * _NW) * _CH
            pltpu.sync_copy(ii_hbm.at[pl.ds(base, _CH)], ii_v)
            pltpu.sync_copy(jj_hbm.at[pl.ds(base, _CH)], jj_v)
            pltpu.async_copy(h_hbm.at[ii_v], bhi, gsem)
            pltpu.async_copy(h_hbm.at[jj_v], bhj, gsem)
            pltpu.async_copy(xp_hbm.at[ii_v], bxi, gsem)
            pltpu.async_copy(xp_hbm.at[jj_v], bxj, gsem)

    def drain_gather_fire_write(t, b):
        ii_v, jj_v, bhi, bhj, bxi, bxj, gsem, wsem = bufs[b]

        @pl.when(valid(t))
        def _():
            base = (wid + t * _NW) * _CH
            pltpu.make_async_copy(h_hbm.at[ii_v], bhi, gsem).wait()
            pltpu.make_async_copy(h_hbm.at[jj_v], bhj, gsem).wait()
            pltpu.make_async_copy(xp_hbm.at[ii_v], bxi, gsem).wait()
            pltpu.make_async_copy(xp_hbm.at[jj_v], bxj, gsem).wait()
            pltpu.async_copy(bhi, hi_hbm.at[pl.ds(base, _CH)], wsem)
            pltpu.async_copy(bhj, hj_hbm.at[pl.ds(base, _CH)], wsem)
            pltpu.async_copy(bxi, xi_hbm.at[pl.ds(base, _CH)], wsem)
            pltpu.async_copy(bxj, xj_hbm.at[pl.ds(base, _CH)], wsem)

    def drain_write(t, b):
        _, _, bhi, bhj, bxi, bxj, _, wsem = bufs[b]

        @pl.when(valid(t))
        def _():
            base = (wid + t * _NW) * _CH
            pltpu.make_async_copy(bhi, hi_hbm.at[pl.ds(base, _CH)], wsem).wait()
            pltpu.make_async_copy(bhj, hj_hbm.at[pl.ds(base, _CH)], wsem).wait()
            pltpu.make_async_copy(bxi, xi_hbm.at[pl.ds(base, _CH)], wsem).wait()
            pltpu.make_async_copy(bxj, xj_hbm.at[pl.ds(base, _CH)], wsem).wait()

    def pair(i):
        t0 = 2 * i
        t1 = 2 * i + 1
        drain_write(t0 - 2, 0)
        fire_gather(t0, 0)
        drain_gather_fire_write(t1 - 2, 1)
        drain_write(t1 - 2, 1)
        fire_gather(t1, 1)
        drain_gather_fire_write(t0, 0)

    def epilogue():
        last = 2 * _GPAIRS - 1
        drain_gather_fire_write(last, 1)
        drain_write(last - 1, 0)
        drain_write(last, 1)

    return pair, epilogue


_GATHER_SCRATCH = [
    pltpu.VMEM((_CH,), jnp.int32),
    pltpu.VMEM((_CH,), jnp.int32),
    pltpu.VMEM((_CH,), jnp.int32),
    pltpu.VMEM((_CH,), jnp.int32),
    pltpu.VMEM((_CH, _ND), jnp.float32),
    pltpu.VMEM((_CH, _ND), jnp.float32),
    pltpu.VMEM((_CH, _XP), jnp.float32),
    pltpu.VMEM((_CH, _XP), jnp.float32),
    pltpu.VMEM((_CH, _ND), jnp.float32),
    pltpu.VMEM((_CH, _ND), jnp.float32),
    pltpu.VMEM((_CH, _XP), jnp.float32),
    pltpu.VMEM((_CH, _XP), jnp.float32),
    pltpu.SemaphoreType.DMA,
    pltpu.SemaphoreType.DMA,
    pltpu.SemaphoreType.DMA,
    pltpu.SemaphoreType.DMA,
]


def _gather_bufs(scratch):
    (ii0, jj0, ii1, jj1, bhi0, bhj0, bxi0, bxj0,
     bhi1, bhj1, bxi1, bxj1, gsem0, gsem1, wsem0, wsem1) = scratch
    return ((ii0, jj0, bhi0, bhj0, bxi0, bxj0, gsem0, wsem0),
            (ii1, jj1, bhi1, bhj1, bxi1, bxj1, gsem1, wsem1))


def _sc_gather_body(h_hbm, xp_hbm, ii_hbm, jj_hbm,
                    hi_hbm, hj_hbm, xi_hbm, xj_hbm, *scratch):
    c = lax.axis_index("c")
    s = lax.axis_index("s")
    wid = s * _NC + c
    pair, epilogue = _mk_gather_pipe(
        h_hbm, xp_hbm, ii_hbm, jj_hbm, hi_hbm, hj_hbm, xi_hbm, xj_hbm,
        _gather_bufs(scratch), wid)

    def body(i, carry):
        pair(i)
        return carry

    lax.fori_loop(0, _GPAIRS, body, 0)
    epilogue()


@functools.lru_cache(maxsize=None)
def _gather_kernel():
    return pl.kernel(
        _sc_gather_body,
        out_type=(
            jax.ShapeDtypeStruct((_ES, _ND), jnp.float32),
            jax.ShapeDtypeStruct((_ES, _ND), jnp.float32),
            jax.ShapeDtypeStruct((_ES, _XP), jnp.float32),
            jax.ShapeDtypeStruct((_ES, _XP), jnp.float32),
        ),
        mesh=_sc_mesh(),
        compiler_params=pltpu.CompilerParams(use_tc_tiling_on_sc=False),
        scratch_types=list(_GATHER_SCRATCH),
    )


def _gather_call(h, xp, ii, jj):
    return _gather_kernel()(h, xp, ii, jj)


def _mk_scatter_pipe(mlo_hbm, mhi_hbm, wd_hbm, ii_hbm, bufs, shm, shx, c, s):
    def valid(t):
        return (t >= 0) & (s + t * _NS < _NCHS)

    def fire_read(t, b):
        idx_v, mb, wb, rsem, _ = bufs[b]
        base = (s + t * _NS) * _CH

        @pl.when(valid(t))
        def _():
            pltpu.async_copy(ii_hbm.at[pl.ds(base, _CH)], idx_v, rsem)

        @pl.when(valid(t) & (c == 0))
        def _():
            pltpu.async_copy(mlo_hbm.at[pl.ds(base, _CH)], mb, rsem)
            pltpu.async_copy(wd_hbm.at[pl.ds(base, _CH)], wb, rsem)

        @pl.when(valid(t) & (c == 1))
        def _():
            pltpu.async_copy(mhi_hbm.at[pl.ds(base, _CH)], mb, rsem)

    def drain_read_fire_add(t, b):
        idx_v, mb, wb, rsem, ssem = bufs[b]
        base = (s + t * _NS) * _CH

        @pl.when(valid(t))
        def _():
            pltpu.make_async_copy(ii_hbm.at[pl.ds(base, _CH)], idx_v,
                                  rsem).wait()

        @pl.when(valid(t) & (c == 0))
        def _():
            pltpu.make_async_copy(mlo_hbm.at[pl.ds(base, _CH)], mb,
                                  rsem).wait()
            pltpu.make_async_copy(wd_hbm.at[pl.ds(base, _CH)], wb,
                                  rsem).wait()
            pltpu.async_copy(mb, shm.at[idx_v], ssem, add=True)
            pltpu.async_copy(wb, shx.at[idx_v], ssem, add=True)

        @pl.when(valid(t) & (c == 1))
        def _():
            pltpu.make_async_copy(mhi_hbm.at[pl.ds(base, _CH)], mb,
                                  rsem).wait()
            pltpu.async_copy(mb, shm.at[idx_v], ssem, add=True)

    def drain_add(t, b):
        idx_v, mb, wb, _, ssem = bufs[b]

        @pl.when(valid(t) & (c == 0))
        def _():
            pltpu.make_async_copy(mb, shm.at[idx_v], ssem).wait()
            pltpu.make_async_copy(wb, shx.at[idx_v], ssem).wait()

        @pl.when(valid(t) & (c == 1))
        def _():
            pltpu.make_async_copy(mb, shm.at[idx_v], ssem).wait()

    def pair(i):
        t0 = 2 * i
        t1 = 2 * i + 1
        drain_add(t0 - 2, 0)
        fire_read(t0, 0)
        drain_read_fire_add(t1 - 2, 1)
        drain_add(t1 - 2, 1)
        fire_read(t1, 1)
        drain_read_fire_add(t0, 0)

    def epilogue():
        last = 2 * _SPAIRS - 1
        drain_read_fire_add(last, 1)
        drain_add(last - 1, 0)
        drain_add(last, 1)

    return pair, epilogue


_SCATTER_SCRATCH = [
    pltpu.VMEM((_CH,), jnp.int32),
    pltpu.VMEM((_CH,), jnp.int32),
    pltpu.VMEM((_CH, _ND), jnp.float32),
    pltpu.VMEM((_CH, _ND), jnp.float32),
    pltpu.VMEM((_CH, _XP), jnp.float32),
    pltpu.VMEM((_CH, _XP), jnp.float32),
    pltpu.VMEM_SHARED((_N, _ND), jnp.float32),
    pltpu.VMEM_SHARED((_N, _XP), jnp.float32),
    pltpu.SemaphoreType.DMA,
    pltpu.SemaphoreType.DMA,
    pltpu.SemaphoreType.DMA,
    pltpu.SemaphoreType.DMA,
]


def _scatter_bufs(scratch):
    idx0, idx1, mb0, mb1, wb0, wb1, shm, shx, rs0, rs1, ss0, ss1 = scratch
    bufs = ((idx0, mb0, wb0, rs0, ss0), (idx1, mb1, wb1, rs1, ss1))
    return bufs, shm, shx


def _acc_init(ilo_hbm, ihi_hbm, ix_hbm, shm, shx, c, rows):
    @pl.when(c == 0)
    def _():
        pltpu.sync_copy(ilo_hbm.at[rows], shm.at[rows])
        pltpu.sync_copy(ix_hbm.at[rows], shx.at[rows])

    @pl.when(c == 1)
    def _():
        pltpu.sync_copy(ihi_hbm.at[rows], shm.at[rows])


def _acc_flush(silo_hbm, sihi_hbm, xacc_hbm, shm, shx, c, rows):
    @pl.when(c == 0)
    def _():
        pltpu.sync_copy(shm.at[rows], silo_hbm.at[rows])
        pltpu.sync_copy(shx.at[rows], xacc_hbm.at[rows])

    @pl.when(c == 1)
    def _():
        pltpu.sync_copy(shm.at[rows], sihi_hbm.at[rows])


def _sc_scatter_body(mlo_hbm, mhi_hbm, wd_hbm, ii_hbm, ilo_hbm, ihi_hbm,
                     ix_hbm,
                     silo_hbm, sihi_hbm, xacc_hbm, *scratch):
    c = lax.axis_index("c")
    s = lax.axis_index("s")
    rows = pl.ds(s * _NPS, _NPS)
    bufs, shm, shx = _scatter_bufs(scratch)

    _acc_init(ilo_hbm, ihi_hbm, ix_hbm, shm, shx, c, rows)
    plsc.subcore_barrier()

    pair, epilogue = _mk_scatter_pipe(
        mlo_hbm, mhi_hbm, wd_hbm, ii_hbm, bufs, shm, shx, c, s)

    def body(i, carry):
        pair(i)
        return carry

    lax.fori_loop(0, _SPAIRS, body, 0)
    epilogue()
    plsc.subcore_barrier()
    _acc_flush(silo_hbm, sihi_hbm, xacc_hbm, shm, shx, c, rows)


@functools.lru_cache(maxsize=None)
def _scatter_kernel():
    return pl.kernel(
        _sc_scatter_body,
        out_type=(
            jax.ShapeDtypeStruct((_N, _ND), jnp.float32),
            jax.ShapeDtypeStruct((_N, _ND), jnp.float32),
            jax.ShapeDtypeStruct((_N, _XP), jnp.float32),
        ),
        mesh=_sc_mesh(),
        compiler_params=pltpu.CompilerParams(use_tc_tiling_on_sc=False),
        scratch_types=list(_SCATTER_SCRATCH),
    )


def _scatter_call(mlo, mhi, wd, ii, ilo, ihi, ix):
    return _scatter_kernel()(mlo, mhi, wd, ii, ilo, ihi, ix)


def _sc_merged_body(h_hbm, xp_hbm, iig_hbm, jjg_hbm,
                    mlo_hbm, mhi_hbm, wd_hbm, iis_hbm,
                    ilo_hbm, ihi_hbm, ix_hbm,
                    hi_hbm, hj_hbm, xi_hbm, xj_hbm,
                    silo_hbm, sihi_hbm, xacc_hbm, *scratch):
    c = lax.axis_index("c")
    s = lax.axis_index("s")
    wid = s * _NC + c
    rows = pl.ds(s * _NPS, _NPS)
    gscratch = scratch[:len(_GATHER_SCRATCH)]
    sscratch = scratch[len(_GATHER_SCRATCH):]
    sbufs, shm, shx = _scatter_bufs(sscratch)

    _acc_init(ilo_hbm, ihi_hbm, ix_hbm, shm, shx, c, rows)
    plsc.subcore_barrier()

    gpair, gepilogue = _mk_gather_pipe(
        h_hbm, xp_hbm, iig_hbm, jjg_hbm, hi_hbm, hj_hbm, xi_hbm, xj_hbm,
        _gather_bufs(gscratch), wid)
    spair, sepilogue = _mk_scatter_pipe(
        mlo_hbm, mhi_hbm, wd_hbm, iis_hbm, sbufs, shm, shx, c, s)

    def body(i, carry):
        gpair(i)
        spair(i)
        return carry

    lax.fori_loop(0, max(_GPAIRS, _SPAIRS), body, 0)
    gepilogue()
    sepilogue()
    plsc.subcore_barrier()
    _acc_flush(silo_hbm, sihi_hbm, xacc_hbm, shm, shx, c, rows)


@functools.lru_cache(maxsize=None)
def _merged_kernel():
    return pl.kernel(
        _sc_merged_body,
        out_type=(
            jax.ShapeDtypeStruct((_ES, _ND), jnp.float32),
            jax.ShapeDtypeStruct((_ES, _ND), jnp.float32),
            jax.ShapeDtypeStruct((_ES, _XP), jnp.float32),
            jax.ShapeDtypeStruct((_ES, _XP), jnp.float32),
            jax.ShapeDtypeStruct((_N, _ND), jnp.float32),
            jax.ShapeDtypeStruct((_N, _ND), jnp.float32),
            jax.ShapeDtypeStruct((_N, _XP), jnp.float32),
        ),
        mesh=_sc_mesh(),
        compiler_params=pltpu.CompilerParams(use_tc_tiling_on_sc=False),
        scratch_types=list(_GATHER_SCRATCH) + list(_SCATTER_SCRATCH),
    )


def _merged_call(h, xp, iig, jjg, mlo, mhi, wd, iis, ilo, ihi, ix):
    return _merged_kernel()(h, xp, iig, jjg, mlo, mhi, wd, iis, ilo, ihi, ix)


def _tc_edge_body(hi, hj, xi, xj, ea,
                  w1, b1, w2, b2, wx1, bx1, wx2,
                  mlo_o, mhi_o, wd_o):
    f32 = jnp.float32
    di = xi[...] - xj[...]
    d2 = jnp.sum(di * di, axis=1, keepdims=True) + 1e-8
    dist = jnp.sqrt(d2)
    centers = lax.broadcasted_iota(jnp.int32, (1, _NRBF), 1).astype(f32) * (
        _CUTOFF / (_NRBF - 1))
    zz = (dist - centers) * (_NRBF / _CUTOFF)
    rbf = jnp.exp(-0.5 * zz * zz)
    msg = jnp.concatenate([hi[...], hj[...], rbf, ea[...]], axis=1)
    pre = jnp.dot(msg, w1[...], preferred_element_type=f32) + b1[...]
    m = _silu(pre)
    m = _silu(jnp.dot(m, w2[...], preferred_element_type=f32) + b2[...])
    t = _silu(jnp.dot(m, wx1[...], preferred_element_type=f32) + bx1[...])
    cw = jnp.dot(t, wx2[...], preferred_element_type=f32)
    mlo_o[...] = m[:, :_ND]
    mhi_o[...] = m[:, _ND:]
    wd_o[...] = di * cw


def _edge_call(hi, hj, xi, xj, ea, w1, b1, w2, b2, wx1, bx1, wx2):
    grid = (_ES // _BE,)
    row = lambda i: (i, 0)
    full = lambda i: (0, 0)
    return pl.pallas_call(
        _tc_edge_body,
        grid=grid,
        in_specs=[
            pl.BlockSpec((_BE, _ND), row),
            pl.BlockSpec((_BE, _ND), row),
            pl.BlockSpec((_BE, _XP), row),
            pl.BlockSpec((_BE, _XP), row),
            pl.BlockSpec((_BE, _ED), row),
            pl.BlockSpec((2 * _ND + _NRBF + _ED, _HD), full),
            pl.BlockSpec((1, _HD), full),
            pl.BlockSpec((_HD, _HD), full),
            pl.BlockSpec((1, _HD), full),
            pl.BlockSpec((_HD, _HD), full),
            pl.BlockSpec((1, _HD), full),
            pl.BlockSpec((_HD, 1), full),
        ],
        out_specs=[
            pl.BlockSpec((_BE, _ND), row),
            pl.BlockSpec((_BE, _ND), row),
            pl.BlockSpec((_BE, _XP), row),
        ],
        out_shape=[
            jax.ShapeDtypeStruct((_ES, _ND), jnp.float32),
            jax.ShapeDtypeStruct((_ES, _ND), jnp.float32),
            jax.ShapeDtypeStruct((_ES, _XP), jnp.float32),
        ],
        compiler_params=pltpu.CompilerParams(
            dimension_semantics=("arbitrary",),
        ),
    )(hi, hj, xi, xj, ea, w1, b1, w2, b2, wx1, bx1, wx2)


def _tc_node_body(h, mlo, mhi, xp, xacc,
                  wh1, bh1, wh2, bh2, g, b,
                  hn_o, xp_o):
    f32 = jnp.float32
    hv = h[...]
    cat = jnp.concatenate([hv, mlo[...], mhi[...]], axis=1)
    pre = jnp.dot(cat, wh1[...], preferred_element_type=f32) + bh1[...]
    u = jnp.dot(_silu(pre), wh2[...], preferred_element_type=f32) + bh2[...]
    hn = hv + u
    mu = jnp.mean(hn, axis=1, keepdims=True)
    var = jnp.mean((hn - mu) * (hn - mu), axis=1, keepdims=True)
    hn_o[...] = (hn - mu) * lax.rsqrt(var + 1e-5) * g[...] + b[...]
    xp_o[...] = xp[...] + xacc[...]


def _node_call(h, mlo, mhi, xp, xacc, wh1, bh1, wh2, bh2, g, b):
    grid = (_N // _BN,)
    row = lambda i: (i, 0)
    full = lambda i: (0, 0)
    return pl.pallas_call(
        _tc_node_body,
        grid=grid,
        in_specs=[
            pl.BlockSpec((_BN, _ND), row),
            pl.BlockSpec((_BN, _ND), row),
            pl.BlockSpec((_BN, _ND), row),
            pl.BlockSpec((_BN, _XP), row),
            pl.BlockSpec((_BN, _XP), row),
            pl.BlockSpec((_ND + _HD, _HD), full),
            pl.BlockSpec((1, _HD), full),
            pl.BlockSpec((_HD, _ND), full),
            pl.BlockSpec((1, _ND), full),
            pl.BlockSpec((1, _ND), full),
            pl.BlockSpec((1, _ND), full),
        ],
        out_specs=[
            pl.BlockSpec((_BN, _ND), row),
            pl.BlockSpec((_BN, _XP), row),
        ],
        out_shape=[
            jax.ShapeDtypeStruct((_N, _ND), jnp.float32),
            jax.ShapeDtypeStruct((_N, _XP), jnp.float32),
        ],
        compiler_params=pltpu.CompilerParams(
            dimension_semantics=("arbitrary",),
        ),
    )(h, mlo, mhi, xp, xacc, wh1, bh1, wh2, bh2, g, b)


def kernel(h, x, edge_index, edge_attr, params):
    ei = edge_index.astype(jnp.int32)
    iis = [lax.slice(ei[1], (k * _ES,), ((k + 1) * _ES,)) for k in range(_S)]
    jjs = [lax.slice(ei[0], (k * _ES,), ((k + 1) * _ES,)) for k in range(_S)]
    eas = [lax.slice(edge_attr, (k * _ES, 0), ((k + 1) * _ES, _ED))
           for k in range(_S)]
    xp = jnp.pad(x.astype(jnp.float32), ((0, 0), (0, _XP - 3)))
    z = jnp.zeros((_N, _ND), jnp.float32)
    zx = jnp.zeros((_N, _XP), jnp.float32)
    for p in params:
        silo, sihi, xacc = z, z, zx
        ew = (p["e1"]["w"], p["e1"]["b"][None],
              p["e2"]["w"], p["e2"]["b"][None],
              p["x1"]["w"], p["x1"]["b"][None], p["x2"]["w"])
        g = [None] * _S
        g[0] = _gather_call(h, xp, iis[0], jjs[0])
        g[1] = _gather_call(h, xp, iis[1], jjs[1])
        for k in range(_S):
            mlo, mhi, wd = _edge_call(*g[k], eas[k], *ew)
            if k + 2 < _S:
                (*g[k + 2], silo, sihi, xacc) = _merged_call(
                    h, xp, iis[k + 2], jjs[k + 2],
                    mlo, mhi, wd, iis[k], silo, sihi, xacc)
                g[k + 2] = tuple(g[k + 2])
            else:
                silo, sihi, xacc = _scatter_call(
                    mlo, mhi, wd, iis[k], silo, sihi, xacc)
        h, xp = _node_call(
            h, silo, sihi, xp, xacc,
            p["h1"]["w"], p["h1"]["b"][None],
            p["h2"]["w"], p["h2"]["b"][None], p["ln_g"][None], p["ln_b"][None],
        )
    return (h, xp[:, :3])


# speedup vs baseline: 1.0470x; 1.0470x over previous
import functools

import jax
import jax.numpy as jnp
from jax import lax
from jax.experimental import pallas as pl
from jax.experimental.pallas import tpu as pltpu
from jax.experimental.pallas import tpu_sc as plsc

_N = 10000
_E = 320000
_ND = 128
_HD = 256
_ED = 16
_NRBF = 16
_CUTOFF = 10.0
_XP = 16

_S = 4
_ES = _E // _S
_CH = 128
_NCHS = _ES // _CH
_NC = 2
_NS = 16
_NW = _NC * _NS
_NPS = _N // _NS

_BE = 1600
_BN = 1000

_GTRIPS = (_NCHS + _NW - 1) // _NW
_GPAIRS = (_GTRIPS + 1) // 2
_STRIPS = (_NCHS + _NS - 1) // _NS
_SPAIRS = (_STRIPS + 1) // 2


@functools.lru_cache(maxsize=None)
def _sc_mesh():
    return plsc.VectorSubcoreMesh(
        core_axis_name="c", subcore_axis_name="s", num_cores=_NC, num_subcores=_NS
    )


def _silu(v):
    return v * jax.nn.sigmoid(v)


def _mk_gather_pipe(h_hbm, xp_hbm, ii_hbm, jj_hbm,
                    hi_hbm, hj_hbm, xi_hbm, xj_hbm, bufs, wid):
    def valid(t):
        return (t >= 0) & (wid + t * _NW < _NCHS)

    def fire_gather(t, b):
        ii_v, jj_v, bhi, bhj, bxi, bxj, gsem, _ = bufs[b]

        @pl.when(valid(t))
        def _():
            base = (wid + t * _NW) * _CH
            pltpu.sync_copy(ii_hbm.at[pl.ds(base, _CH)], ii_v)
            pltpu.sync_copy(jj_hbm.at[pl.ds(base, _CH)], jj_v)
            pltpu.async_copy(h_hbm.at[ii_v], bhi, gsem)
            pltpu.async_copy(h_hbm.at[jj_v], bhj, gsem)
            pltpu.async_copy(xp_hbm.at[ii_v], bxi, gsem)
            pltpu.async_copy(xp_hbm.at[jj_v], bxj, gsem)

    def drain_gather_fire_write(t, b):
        ii_v, jj_v, bhi, bhj, bxi, bxj, gsem, wsem = bufs[b]

        @pl.when(valid(t))
        def _():
            base = (wid + t * _NW) * _CH
            pltpu.make_async_copy(h_hbm.at[ii_v], bhi, gsem).wait()
            pltpu.make_async_copy(h_hbm.at[jj_v], bhj, gsem).wait()
            pltpu.make_async_copy(xp_hbm.at[ii_v], bxi, gsem).wait()
            pltpu.make_async_copy(xp_hbm.at[jj_v], bxj, gsem).wait()
            pltpu.async_copy(bhi, hi_hbm.at[pl.ds(base, _CH)], wsem)
            pltpu.async_copy(bhj, hj_hbm.at[pl.ds(base, _CH)], wsem)
            pltpu.async_copy(bxi, xi_hbm.at[pl.ds(base, _CH)], wsem)
            pltpu.async_copy(bxj, xj_hbm.at[pl.ds(base, _CH)], wsem)

    def drain_write(t, b):
        _, _, bhi, bhj, bxi, bxj, _, wsem = bufs[b]

        @pl.when(valid(t))
        def _():
            base = (wid + t * _NW) * _CH
            pltpu.make_async_copy(bhi, hi_hbm.at[pl.ds(base, _CH)], wsem).wait()
            pltpu.make_async_copy(bhj, hj_hbm.at[pl.ds(base, _CH)], wsem).wait()
            pltpu.make_async_copy(bxi, xi_hbm.at[pl.ds(base, _CH)], wsem).wait()
            pltpu.make_async_copy(bxj, xj_hbm.at[pl.ds(base, _CH)], wsem).wait()

    def pair(i):
        t0 = 2 * i
        t1 = 2 * i + 1
        drain_write(t0 - 2, 0)
        fire_gather(t0, 0)
        drain_gather_fire_write(t1 - 2, 1)
        drain_write(t1 - 2, 1)
        fire_gather(t1, 1)
        drain_gather_fire_write(t0, 0)

    def epilogue():
        last = 2 * _GPAIRS - 1
        drain_gather_fire_write(last, 1)
        drain_write(last - 1, 0)
        drain_write(last, 1)

    return pair, epilogue


_GATHER_SCRATCH = [
    pltpu.VMEM((_CH,), jnp.int32),
    pltpu.VMEM((_CH,), jnp.int32),
    pltpu.VMEM((_CH,), jnp.int32),
    pltpu.VMEM((_CH,), jnp.int32),
    pltpu.VMEM((_CH, _ND), jnp.float32),
    pltpu.VMEM((_CH, _ND), jnp.float32),
    pltpu.VMEM((_CH, _XP), jnp.float32),
    pltpu.VMEM((_CH, _XP), jnp.float32),
    pltpu.VMEM((_CH, _ND), jnp.float32),
    pltpu.VMEM((_CH, _ND), jnp.float32),
    pltpu.VMEM((_CH, _XP), jnp.float32),
    pltpu.VMEM((_CH, _XP), jnp.float32),
    pltpu.SemaphoreType.DMA,
    pltpu.SemaphoreType.DMA,
    pltpu.SemaphoreType.DMA,
    pltpu.SemaphoreType.DMA,
]


def _gather_bufs(scratch):
    (ii0, jj0, ii1, jj1, bhi0, bhj0, bxi0, bxj0,
     bhi1, bhj1, bxi1, bxj1, gsem0, gsem1, wsem0, wsem1) = scratch
    return ((ii0, jj0, bhi0, bhj0, bxi0, bxj0, gsem0, wsem0),
            (ii1, jj1, bhi1, bhj1, bxi1, bxj1, gsem1, wsem1))


def _sc_gather_body(h_hbm, xp_hbm, ii_hbm, jj_hbm,
                    hi_hbm, hj_hbm, xi_hbm, xj_hbm, *scratch):
    c = lax.axis_index("c")
    s = lax.axis_index("s")
    wid = s * _NC + c
    pair, epilogue = _mk_gather_pipe(
        h_hbm, xp_hbm, ii_hbm, jj_hbm, hi_hbm, hj_hbm, xi_hbm, xj_hbm,
        _gather_bufs(scratch), wid)

    def body(i, carry):
        pair(i)
        return carry

    lax.fori_loop(0, _GPAIRS, body, 0)
    epilogue()


@functools.lru_cache(maxsize=None)
def _gather_kernel():
    return pl.kernel(
        _sc_gather_body,
        out_type=(
            jax.ShapeDtypeStruct((_ES, _ND), jnp.float32),
            jax.ShapeDtypeStruct((_ES, _ND), jnp.float32),
            jax.ShapeDtypeStruct((_ES, _XP), jnp.float32),
            jax.ShapeDtypeStruct((_ES, _XP), jnp.float32),
        ),
        mesh=_sc_mesh(),
        compiler_params=pltpu.CompilerParams(use_tc_tiling_on_sc=False),
        scratch_types=list(_GATHER_SCRATCH),
    )


def _gather_call(h, xp, ii, jj):
    return _gather_kernel()(h, xp, ii, jj)


def _mk_scatter_pipe(mlo_hbm, mhi_hbm, wd_hbm, ii_hbm, bufs, shm, shx, c, s):
    def valid(t):
        return (t >= 0) & (s + t * _NS < _NCHS)

    def fire_read(t, b):
        idx_v, mb, wb, rsem, _ = bufs[b]
        base = (s + t * _NS) * _CH

        @pl.when(valid(t))
        def _():
            pltpu.async_copy(ii_hbm.at[pl.ds(base, _CH)], idx_v, rsem)

        @pl.when(valid(t) & (c == 0))
        def _():
            pltpu.async_copy(mlo_hbm.at[pl.ds(base, _CH)], mb, rsem)
            pltpu.async_copy(wd_hbm.at[pl.ds(base, _CH)], wb, rsem)

        @pl.when(valid(t) & (c == 1))
        def _():
            pltpu.async_copy(mhi_hbm.at[pl.ds(base, _CH)], mb, rsem)

    def drain_read_fire_add(t, b):
        idx_v, mb, wb, rsem, ssem = bufs[b]
        base = (s + t * _NS) * _CH

        @pl.when(valid(t))
        def _():
            pltpu.make_async_copy(ii_hbm.at[pl.ds(base, _CH)], idx_v,
                                  rsem).wait()

        @pl.when(valid(t) & (c == 0))
        def _():
            pltpu.make_async_copy(mlo_hbm.at[pl.ds(base, _CH)], mb,
                                  rsem).wait()
            pltpu.make_async_copy(wd_hbm.at[pl.ds(base, _CH)], wb,
                                  rsem).wait()
            pltpu.async_copy(mb, shm.at[idx_v], ssem, add=True)
            pltpu.async_copy(wb, shx.at[idx_v], ssem, add=True)

        @pl.when(valid(t) & (c == 1))
        def _():
            pltpu.make_async_copy(mhi_hbm.at[pl.ds(base, _CH)], mb,
                                  rsem).wait()
            pltpu.async_copy(mb, shm.at[idx_v], ssem, add=True)

    def drain_add(t, b):
        idx_v, mb, wb, _, ssem = bufs[b]

        @pl.when(valid(t) & (c == 0))
        def _():
            pltpu.make_async_copy(mb, shm.at[idx_v], ssem).wait()
            pltpu.make_async_copy(wb, shx.at[idx_v], ssem).wait()

        @pl.when(valid(t) & (c == 1))
        def _():
            pltpu.make_async_copy(mb, shm.at[idx_v], ssem).wait()

    def pair(i):
        t0 = 2 * i
        t1 = 2 * i + 1
        drain_add(t0 - 2, 0)
        fire_read(t0, 0)
        drain_read_fire_add(t1 - 2, 1)
        drain_add(t1 - 2, 1)
        fire_read(t1, 1)
        drain_read_fire_add(t0, 0)

    def epilogue():
        last = 2 * _SPAIRS - 1
        drain_read_fire_add(last, 1)
        drain_add(last - 1, 0)
        drain_add(last, 1)

    return pair, epilogue


_SCATTER_SCRATCH = [
    pltpu.VMEM((_CH,), jnp.int32),
    pltpu.VMEM((_CH,), jnp.int32),
    pltpu.VMEM((_CH, _ND), jnp.float32),
    pltpu.VMEM((_CH, _ND), jnp.float32),
    pltpu.VMEM((_CH, _XP), jnp.float32),
    pltpu.VMEM((_CH, _XP), jnp.float32),
    pltpu.VMEM_SHARED((_N, _ND), jnp.float32),
    pltpu.VMEM_SHARED((_N, _XP), jnp.float32),
    pltpu.SemaphoreType.DMA,
    pltpu.SemaphoreType.DMA,
    pltpu.SemaphoreType.DMA,
    pltpu.SemaphoreType.DMA,
]


def _scatter_bufs(scratch):
    idx0, idx1, mb0, mb1, wb0, wb1, shm, shx, rs0, rs1, ss0, ss1 = scratch
    bufs = ((idx0, mb0, wb0, rs0, ss0), (idx1, mb1, wb1, rs1, ss1))
    return bufs, shm, shx


def _acc_init(ilo_hbm, ihi_hbm, ix_hbm, shm, shx, c, rows):
    @pl.when(c == 0)
    def _():
        pltpu.sync_copy(ilo_hbm.at[rows], shm.at[rows])
        pltpu.sync_copy(ix_hbm.at[rows], shx.at[rows])

    @pl.when(c == 1)
    def _():
        pltpu.sync_copy(ihi_hbm.at[rows], shm.at[rows])


def _acc_flush(silo_hbm, sihi_hbm, xacc_hbm, shm, shx, c, rows):
    @pl.when(c == 0)
    def _():
        pltpu.sync_copy(shm.at[rows], silo_hbm.at[rows])
        pltpu.sync_copy(shx.at[rows], xacc_hbm.at[rows])

    @pl.when(c == 1)
    def _():
        pltpu.sync_copy(shm.at[rows], sihi_hbm.at[rows])


def _sc_scatter_body(mlo_hbm, mhi_hbm, wd_hbm, ii_hbm, ilo_hbm, ihi_hbm,
                     ix_hbm,
                     silo_hbm, sihi_hbm, xacc_hbm, *scratch):
    c = lax.axis_index("c")
    s = lax.axis_index("s")
    rows = pl.ds(s * _NPS, _NPS)
    bufs, shm, shx = _scatter_bufs(scratch)

    _acc_init(ilo_hbm, ihi_hbm, ix_hbm, shm, shx, c, rows)
    plsc.subcore_barrier()

    pair, epilogue = _mk_scatter_pipe(
        mlo_hbm, mhi_hbm, wd_hbm, ii_hbm, bufs, shm, shx, c, s)

    def body(i, carry):
        pair(i)
        return carry

    lax.fori_loop(0, _SPAIRS, body, 0)
    epilogue()
    plsc.subcore_barrier()
    _acc_flush(silo_hbm, sihi_hbm, xacc_hbm, shm, shx, c, rows)


@functools.lru_cache(maxsize=None)
def _scatter_kernel():
    return pl.kernel(
        _sc_scatter_body,
        out_type=(
            jax.ShapeDtypeStruct((_N, _ND), jnp.float32),
            jax.ShapeDtypeStruct((_N, _ND), jnp.float32),
            jax.ShapeDtypeStruct((_N, _XP), jnp.float32),
        ),
        mesh=_sc_mesh(),
        compiler_params=pltpu.CompilerParams(use_tc_tiling_on_sc=False),
        scratch_types=list(_SCATTER_SCRATCH),
    )


def _scatter_call(mlo, mhi, wd, ii, ilo, ihi, ix):
    return _scatter_kernel()(mlo, mhi, wd, ii, ilo, ihi, ix)


def _tc_edge_body(hi, hj, xi, xj, ea,
                  w1, b1, w2, b2, wx1, bx1, wx2,
                  mlo_o, mhi_o, wd_o):
    f32 = jnp.float32
    di = xi[...] - xj[...]
    d2 = jnp.sum(di * di, axis=1, keepdims=True) + 1e-8
    dist = jnp.sqrt(d2)
    centers = lax.broadcasted_iota(jnp.int32, (1, _NRBF), 1).astype(f32) * (
        _CUTOFF / (_NRBF - 1))
    zz = (dist - centers) * (_NRBF / _CUTOFF)
    rbf = jnp.exp(-0.5 * zz * zz)
    msg = jnp.concatenate([hi[...], hj[...], rbf, ea[...]], axis=1)
    pre = jnp.dot(msg, w1[...], preferred_element_type=f32) + b1[...]
    m = _silu(pre)
    m = _silu(jnp.dot(m, w2[...], preferred_element_type=f32) + b2[...])
    t = _silu(jnp.dot(m, wx1[...], preferred_element_type=f32) + bx1[...])
    cw = jnp.dot(t, wx2[...], preferred_element_type=f32)
    mlo_o[...] = m[:, :_ND]
    mhi_o[...] = m[:, _ND:]
    wd_o[...] = di * cw


def _edge_call(hi, hj, xi, xj, ea, w1, b1, w2, b2, wx1, bx1, wx2):
    grid = (_ES // _BE,)
    row = lambda i: (i, 0)
    full = lambda i: (0, 0)
    return pl.pallas_call(
        _tc_edge_body,
        grid=grid,
        in_specs=[
            pl.BlockSpec((_BE, _ND), row),
            pl.BlockSpec((_BE, _ND), row),
            pl.BlockSpec((_BE, _XP), row),
            pl.BlockSpec((_BE, _XP), row),
            pl.BlockSpec((_BE, _ED), row),
            pl.BlockSpec((2 * _ND + _NRBF + _ED, _HD), full),
            pl.BlockSpec((1, _HD), full),
            pl.BlockSpec((_HD, _HD), full),
            pl.BlockSpec((1, _HD), full),
            pl.BlockSpec((_HD, _HD), full),
            pl.BlockSpec((1, _HD), full),
            pl.BlockSpec((_HD, 1), full),
        ],
        out_specs=[
            pl.BlockSpec((_BE, _ND), row),
            pl.BlockSpec((_BE, _ND), row),
            pl.BlockSpec((_BE, _XP), row),
        ],
        out_shape=[
            jax.ShapeDtypeStruct((_ES, _ND), jnp.float32),
            jax.ShapeDtypeStruct((_ES, _ND), jnp.float32),
            jax.ShapeDtypeStruct((_ES, _XP), jnp.float32),
        ],
        compiler_params=pltpu.CompilerParams(
            dimension_semantics=("arbitrary",),
        ),
    )(hi, hj, xi, xj, ea, w1, b1, w2, b2, wx1, bx1, wx2)


def _tc_node_body(h, mlo, mhi, xp, xacc,
                  wh1, bh1, wh2, bh2, g, b,
                  hn_o, xp_o):
    f32 = jnp.float32
    hv = h[...]
    cat = jnp.concatenate([hv, mlo[...], mhi[...]], axis=1)
    pre = jnp.dot(cat, wh1[...], preferred_element_type=f32) + bh1[...]
    u = jnp.dot(_silu(pre), wh2[...], preferred_element_type=f32) + bh2[...]
    hn = hv + u
    mu = jnp.mean(hn, axis=1, keepdims=True)
    var = jnp.mean((hn - mu) * (hn - mu), axis=1, keepdims=True)
    hn_o[...] = (hn - mu) * lax.rsqrt(var + 1e-5) * g[...] + b[...]
    xp_o[...] = xp[...] + xacc[...]


def _node_call(h, mlo, mhi, xp, xacc, wh1, bh1, wh2, bh2, g, b):
    grid = (_N // _BN,)
    row = lambda i: (i, 0)
    full = lambda i: (0, 0)
    return pl.pallas_call(
        _tc_node_body,
        grid=grid,
        in_specs=[
            pl.BlockSpec((_BN, _ND), row),
            pl.BlockSpec((_BN, _ND), row),
            pl.BlockSpec((_BN, _ND), row),
            pl.BlockSpec((_BN, _XP), row),
            pl.BlockSpec((_BN, _XP), row),
            pl.BlockSpec((_ND + _HD, _HD), full),
            pl.BlockSpec((1, _HD), full),
            pl.BlockSpec((_HD, _ND), full),
            pl.BlockSpec((1, _ND), full),
            pl.BlockSpec((1, _ND), full),
            pl.BlockSpec((1, _ND), full),
        ],
        out_specs=[
            pl.BlockSpec((_BN, _ND), row),
            pl.BlockSpec((_BN, _XP), row),
        ],
        out_shape=[
            jax.ShapeDtypeStruct((_N, _ND), jnp.float32),
            jax.ShapeDtypeStruct((_N, _XP), jnp.float32),
        ],
        compiler_params=pltpu.CompilerParams(
            dimension_semantics=("arbitrary",),
        ),
    )(h, mlo, mhi, xp, xacc, wh1, bh1, wh2, bh2, g, b)


def kernel(h, x, edge_index, edge_attr, params):
    ei = edge_index.astype(jnp.int32)
    iis = [lax.slice(ei[1], (k * _ES,), ((k + 1) * _ES,)) for k in range(_S)]
    jjs = [lax.slice(ei[0], (k * _ES,), ((k + 1) * _ES,)) for k in range(_S)]
    eas = [lax.slice(edge_attr, (k * _ES, 0), ((k + 1) * _ES, _ED))
           for k in range(_S)]
    xp = jnp.pad(x.astype(jnp.float32), ((0, 0), (0, _XP - 3)))
    z = jnp.zeros((_N, _ND), jnp.float32)
    zx = jnp.zeros((_N, _XP), jnp.float32)
    for p in params:
        silo, sihi, xacc = z, z, zx
        ew = (p["e1"]["w"], p["e1"]["b"][None],
              p["e2"]["w"], p["e2"]["b"][None],
              p["x1"]["w"], p["x1"]["b"][None], p["x2"]["w"])
        for k in range(_S):
            hi, hj, xi, xj = _gather_call(h, xp, iis[k], jjs[k])
            mlo, mhi, wd = _edge_call(hi, hj, xi, xj, eas[k], *ew)
            silo, sihi, xacc = _scatter_call(
                mlo, mhi, wd, iis[k], silo, sihi, xacc)
        h, xp = _node_call(
            h, silo, sihi, xp, xacc,
            p["h1"]["w"], p["h1"]["b"][None],
            p["h2"]["w"], p["h2"]["b"][None], p["ln_g"][None], p["ln_b"][None],
        )
    return (h, xp[:, :3])
